# Initial kernel scaffold; baseline (speedup 1.0000x reference)
import jax
import jax.numpy as jnp
from jax.experimental import pallas as pl


def kernel(x, edge_index, edge_attr, W1, root1, b1, gamma, beta, W2, root2, b2):
    raise NotImplementedError("write your pallas kernel here")


# speedup vs baseline: 4.8335x; 4.8335x over previous
import functools

import jax
import jax.numpy as jnp
from jax import lax
from jax.experimental import pallas as pl
from jax.experimental.pallas import tpu as pltpu
from jax.experimental.pallas import tpu_sc as plsc

_NR = 20
_N = 10000
_E = 320000
_D = 128
_RP = 32

_NC = 2
_NS = 16
_EPT = _E // (_NC * _NS)
_CH = 80
_NCHUNK = _EPT // _CH
_RPT = _N // _NS

_BLK = 1000
_NBLK = _N // _BLK


def _i16(v):
    return jnp.full((16,), v, dtype=jnp.int32)


def _f16(v):
    return jnp.full((16,), v, dtype=jnp.float32)


def _a_body(x_ref, w_ref, z_ref):
    z_ref[...] = jnp.dot(x_ref[...], w_ref[0], preferred_element_type=jnp.float32)


def _run_a(x, waug):
    return pl.pallas_call(
        _a_body,
        grid=(_NBLK, _NR + 1),
        in_specs=[
            pl.BlockSpec((_BLK, _D), lambda j, r: (j, 0)),
            pl.BlockSpec((1, _D, _D), lambda j, r: (r, 0, 0)),
        ],
        out_specs=pl.BlockSpec((_BLK, _D), lambda j, r: (r * _NBLK + j, 0)),
        out_shape=jax.ShapeDtypeStruct(((_NR + 1) * _N, _D), jnp.float32),
    )(x, waug)


def _b1_body(srca, dsta, ea, cnt_out, dst_v, rel_v, oh_v, zb_v, acc_s):
    c = lax.axis_index("c")
    s = lax.axis_index("s")

    @pl.loop(0, 125)
    def _zero(r):
        zb_v[r, pl.ds(0, 16)] = _f16(0.0)
        zb_v[r, pl.ds(16, 16)] = _f16(0.0)

    for j in range(5):
        pltpu.sync_copy(zb_v, acc_s.at[pl.ds(s * _RPT + j * 125, 125)])

    @pl.loop(0, _CH)
    def _zoh(r):
        oh_v[r, pl.ds(0, 16)] = _f16(0.0)
        oh_v[r, pl.ds(16, 16)] = _f16(0.0)

    plsc.subcore_barrier()

    tile_base = c * (_NS * _EPT) + s * _EPT

    @pl.loop(0, _NCHUNK)
    def _chunk(i):
        base = tile_base + i * _CH
        pltpu.sync_copy(dsta.at[pl.ds(base, _CH)], dst_v)
        pltpu.sync_copy(ea.at[pl.ds(base, _CH)], rel_v)
        iota = lax.iota(jnp.int32, 16)
        for g in range(_CH // 16):
            e16 = iota + _i16(g * 16)
            rel16 = rel_v[pl.ds(g * 16, 16)]
            plsc.store_scatter(oh_v, [e16, rel16], _f16(1.0))
        pltpu.sync_copy(oh_v, acc_s.at[dst_v], add=True)
        for g in range(_CH // 16):
            e16 = iota + _i16(g * 16)
            rel16 = rel_v[pl.ds(g * 16, 16)]
            plsc.store_scatter(oh_v, [e16, rel16], _f16(0.0))

    plsc.subcore_barrier()
    pltpu.sync_copy(acc_s.at[pl.ds(s * _RPT, _RPT)],
                    cnt_out.at[c, pl.ds(s * _RPT, _RPT)])


def _run_b1(srca, dsta, ea):
    mesh = plsc.VectorSubcoreMesh(core_axis_name="c", subcore_axis_name="s")
    return pl.kernel(
        _b1_body,
        out_type=jax.ShapeDtypeStruct((_NC, _N, _RP), jnp.float32),
        mesh=mesh,
        compiler_params=pltpu.CompilerParams(use_tc_tiling_on_sc=False, needs_layout_passes=False),
        scratch_types=[
            pltpu.VMEM((_CH,), jnp.int32),
            pltpu.VMEM((_CH,), jnp.int32),
            pltpu.VMEM((_CH, _RP), jnp.float32),
            pltpu.VMEM((125, _RP), jnp.float32),
            pltpu.VMEM_SHARED((_N, _RP), jnp.float32),
        ],
    )(srca, dsta, ea)


def _c_body(cnt_ref, r_ref):
    c = cnt_ref[0] + cnt_ref[1]
    r_ref[...] = 1.0 / jnp.maximum(c, 1.0)


def _run_c(cnt2):
    return pl.pallas_call(
        _c_body,
        grid=(_NBLK,),
        in_specs=[pl.BlockSpec((_NC, _BLK, _RP), lambda j: (0, j, 0))],
        out_specs=pl.BlockSpec((_BLK, _RP), lambda j: (j, 0)),
        out_shape=jax.ShapeDtypeStruct((_N, _RP), jnp.float32),
    )(cnt2)


def _b2_body(z, r2d, srca, dsta, ea, agg_out,
             src_v, dst_v, rel_v, gidx_v, wrows_v, wbuf_v, msg_v, zb_v,
             acc_s, sem1, sem2):
    c = lax.axis_index("c")
    s = lax.axis_index("s")

    @pl.loop(0, 125)
    def _zero(r):
        for f in range(8):
            zb_v[r, pl.ds(f * 16, 16)] = _f16(0.0)

    for j in range(5):
        pltpu.sync_copy(zb_v, acc_s.at[pl.ds(s * _RPT + j * 125, 125)])

    plsc.subcore_barrier()

    tile_base = c * (_NS * _EPT) + s * _EPT
    iota = lax.iota(jnp.int32, 16)

    @pl.loop(0, _NCHUNK)
    def _chunk(i):
        base = tile_base + i * _CH
        pltpu.sync_copy(srca.at[pl.ds(base, _CH)], src_v)
        pltpu.sync_copy(dsta.at[pl.ds(base, _CH)], dst_v)
        pltpu.sync_copy(ea.at[pl.ds(base, _CH)], rel_v)
        for g in range(_CH // 16):
            sl = pl.ds(g * 16, 16)
            gidx_v[sl] = rel_v[sl] * _N + src_v[sl]
        pltpu.async_copy(z.at[gidx_v], msg_v, sem1).wait()
        pltpu.async_copy(r2d.at[dst_v], wrows_v, sem2).wait()
        for g in range(_CH // 16):
            sl = pl.ds(g * 16, 16)
            e16 = iota + _i16(g * 16)
            w16 = plsc.load_gather(wrows_v, [e16, rel_v[sl]])
            for k in range(16):
                wb = lax.gather(
                    w16, _i16(k)[:, None],
                    lax.GatherDimensionNumbers(offset_dims=(),
                                               collapsed_slice_dims=(0,),
                                               start_index_map=(0,)),
                    (1,), mode=lax.GatherScatterMode.PROMISE_IN_BOUNDS)
                e = g * 16 + k
                for f in range(8):
                    fsl = pl.ds(f * 16, 16)
                    msg_v[e, fsl] = msg_v[e, fsl] * wb
        pltpu.sync_copy(msg_v, acc_s.at[dst_v], add=True)

    plsc.subcore_barrier()
    pltpu.sync_copy(acc_s.at[pl.ds(s * _RPT, _RPT)],
                    agg_out.at[c, pl.ds(s * _RPT, _RPT)])


def _run_b2(z, r2d, srca, dsta, ea):
    mesh = plsc.VectorSubcoreMesh(core_axis_name="c", subcore_axis_name="s")
    return pl.kernel(
        _b2_body,
        out_type=jax.ShapeDtypeStruct((_NC, _N, _D), jnp.float32),
        mesh=mesh,
        compiler_params=pltpu.CompilerParams(use_tc_tiling_on_sc=False, needs_layout_passes=False),
        scratch_types=[
            pltpu.VMEM((_CH,), jnp.int32),
            pltpu.VMEM((_CH,), jnp.int32),
            pltpu.VMEM((_CH,), jnp.int32),
            pltpu.VMEM((_CH,), jnp.int32),
            pltpu.VMEM((_CH, _RP), jnp.float32),
            pltpu.VMEM((_CH // 16, 16), jnp.float32),
            pltpu.VMEM((_CH, _D), jnp.float32),
            pltpu.VMEM((125, _D), jnp.float32),
            pltpu.VMEM_SHARED((_N, _D), jnp.float32),
            pltpu.SemaphoreType.DMA,
            pltpu.SemaphoreType.DMA,
        ],
    )(z, r2d, srca, dsta, ea)


def _d1_body(agg_ref, z_ref, b1_ref, hpre_ref, st_ref):
    h = agg_ref[0] + agg_ref[1] + z_ref[...] + b1_ref[...]
    hpre_ref[...] = h
    s = jnp.sum(h, axis=0, keepdims=True)
    ss = jnp.sum(h * h, axis=0, keepdims=True)

    @pl.when(pl.program_id(0) == 0)
    def _():
        st_ref[...] = jnp.zeros_like(st_ref)

    st_ref[0:1, :] = st_ref[0:1, :] + s
    st_ref[1:2, :] = st_ref[1:2, :] + ss


def _run_d1(agg1, z, b1r):
    return pl.pallas_call(
        _d1_body,
        grid=(_NBLK,),
        in_specs=[
            pl.BlockSpec((_NC, _BLK, _D), lambda j: (0, j, 0)),
            pl.BlockSpec((_BLK, _D), lambda j: (_NR * _NBLK + j, 0)),
            pl.BlockSpec((1, _D), lambda j: (0, 0)),
        ],
        out_specs=[
            pl.BlockSpec((_BLK, _D), lambda j: (j, 0)),
            pl.BlockSpec((8, _D), lambda j: (0, 0)),
        ],
        out_shape=[
            jax.ShapeDtypeStruct((_N, _D), jnp.float32),
            jax.ShapeDtypeStruct((8, _D), jnp.float32),
        ],
    )(agg1, z, b1r)


def _d2_body(hpre_ref, st_ref, g_ref, b_ref, w_ref, z2_ref):
    inv_n = 1.0 / _N
    m = st_ref[0:1, :] * inv_n
    var = st_ref[1:2, :] * inv_n - m * m
    inv = lax.rsqrt(var + 1e-5)
    h = (hpre_ref[...] - m) * (inv * g_ref[...]) + b_ref[...]
    h = jnp.maximum(h, 0.0)
    z2_ref[...] = jnp.dot(h, w_ref[...], preferred_element_type=jnp.float32)


def _run_d2(hpre, stats, gr, br, wcat):
    return pl.pallas_call(
        _d2_body,
        grid=(_NBLK,),
        in_specs=[
            pl.BlockSpec((_BLK, _D), lambda j: (j, 0)),
            pl.BlockSpec((8, _D), lambda j: (0, 0)),
            pl.BlockSpec((1, _D), lambda j: (0, 0)),
            pl.BlockSpec((1, _D), lambda j: (0, 0)),
            pl.BlockSpec((_D, _RP), lambda j: (0, 0)),
        ],
        out_specs=pl.BlockSpec((_BLK, _RP), lambda j: (j, 0)),
        out_shape=jax.ShapeDtypeStruct((_N, _RP), jnp.float32),
    )(hpre, stats, gr, br, wcat)


def _b4_body(z2, srca, dsta, ea, agg_out,
             src_v, dst_v, rel_v, zrows_v, msg_v, zb_v, acc_s, sem1):
    c = lax.axis_index("c")
    s = lax.axis_index("s")

    @pl.loop(0, 125)
    def _zero(r):
        zb_v[r, pl.ds(0, 16)] = _f16(0.0)
        zb_v[r, pl.ds(16, 16)] = _f16(0.0)

    for j in range(5):
        pltpu.sync_copy(zb_v, acc_s.at[pl.ds(s * _RPT + j * 125, 125)])

    @pl.loop(0, _CH)
    def _zmsg(r):
        msg_v[r, pl.ds(0, 16)] = _f16(0.0)
        msg_v[r, pl.ds(16, 16)] = _f16(0.0)

    plsc.subcore_barrier()

    tile_base = c * (_NS * _EPT) + s * _EPT
    iota = lax.iota(jnp.int32, 16)

    @pl.loop(0, _NCHUNK)
    def _chunk(i):
        base = tile_base + i * _CH
        pltpu.sync_copy(srca.at[pl.ds(base, _CH)], src_v)
        pltpu.sync_copy(dsta.at[pl.ds(base, _CH)], dst_v)
        pltpu.sync_copy(ea.at[pl.ds(base, _CH)], rel_v)
        pltpu.async_copy(z2.at[src_v], zrows_v, sem1).wait()
        for g in range(_CH // 16):
            sl = pl.ds(g * 16, 16)
            e16 = iota + _i16(g * 16)
            rel16 = rel_v[sl]
            val16 = plsc.load_gather(zrows_v, [e16, rel16])
            plsc.store_scatter(msg_v, [e16, rel16], val16)
        pltpu.sync_copy(msg_v, acc_s.at[dst_v], add=True)
        for g in range(_CH // 16):
            sl = pl.ds(g * 16, 16)
            e16 = iota + _i16(g * 16)
            plsc.store_scatter(msg_v, [e16, rel_v[sl]], _f16(0.0))

    plsc.subcore_barrier()
    pltpu.sync_copy(acc_s.at[pl.ds(s * _RPT, _RPT)],
                    agg_out.at[c, pl.ds(s * _RPT, _RPT)])


def _run_b4(z2, srca, dsta, ea):
    mesh = plsc.VectorSubcoreMesh(core_axis_name="c", subcore_axis_name="s")
    return pl.kernel(
        _b4_body,
        out_type=jax.ShapeDtypeStruct((_NC, _N, _RP), jnp.float32),
        mesh=mesh,
        compiler_params=pltpu.CompilerParams(use_tc_tiling_on_sc=False, needs_layout_passes=False),
        scratch_types=[
            pltpu.VMEM((_CH,), jnp.int32),
            pltpu.VMEM((_CH,), jnp.int32),
            pltpu.VMEM((_CH,), jnp.int32),
            pltpu.VMEM((_CH, _RP), jnp.float32),
            pltpu.VMEM((_CH, _RP), jnp.float32),
            pltpu.VMEM((125, _RP), jnp.float32),
            pltpu.VMEM_SHARED((_N, _RP), jnp.float32),
            pltpu.SemaphoreType.DMA,
        ],
    )(z2, srca, dsta, ea)


def _d3_body(agg_ref, r_ref, z2_ref, b2_ref, out_ref):
    t = (agg_ref[0] + agg_ref[1]) * r_ref[...]
    tsum = jnp.sum(t, axis=1, keepdims=True)
    out_ref[...] = tsum + z2_ref[:, _NR:_NR + 1] + b2_ref[...]


def _run_d3(agg2, r2d, z2cat, b2r):
    return pl.pallas_call(
        _d3_body,
        grid=(_NBLK,),
        in_specs=[
            pl.BlockSpec((_NC, _BLK, _RP), lambda j: (0, j, 0)),
            pl.BlockSpec((_BLK, _RP), lambda j: (j, 0)),
            pl.BlockSpec((_BLK, _RP), lambda j: (j, 0)),
            pl.BlockSpec((1, 1), lambda j: (0, 0)),
        ],
        out_specs=pl.BlockSpec((_BLK, 1), lambda j: (j, 0)),
        out_shape=jax.ShapeDtypeStruct((_N, 1), jnp.float32),
    )(agg2, r2d, z2cat, b2r)


@jax.jit
def kernel(x, edge_index, edge_attr, W1, root1, b1, gamma, beta, W2, root2, b2):
    ei = edge_index.astype(jnp.int32)
    ea = edge_attr.astype(jnp.int32)

    waug = jnp.concatenate([W1, root1[None]], axis=0)
    wcat = jnp.concatenate(
        [jnp.transpose(W2[:, :, 0]), root2,
         jnp.zeros((_D, _RP - _NR - 1), jnp.float32)], axis=1)

    z = _run_a(x, waug)
    srca = ei[0]
    dsta = ei[1]
    cnt2 = _run_b1(srca, dsta, ea)
    r2d = _run_c(cnt2)
    agg1 = _run_b2(z, r2d, srca, dsta, ea)
    hpre, stats = _run_d1(agg1, z, b1[None])
    z2cat = _run_d2(hpre, stats, gamma[None], beta[None], wcat)
    agg2 = _run_b4(z2cat, srca, dsta, ea)
    out = _run_d3(agg2, r2d, z2cat, b2[None])
    return out


# speedup vs baseline: 6.7463x; 1.3958x over previous
import functools

import jax
import jax.numpy as jnp
from jax import lax
from jax.experimental import pallas as pl
from jax.experimental.pallas import tpu as pltpu
from jax.experimental.pallas import tpu_sc as plsc

_NR = 20
_N = 10000
_E = 320000
_D = 128
_RP = 32

_NC = 2
_NS = 16
_EPT = _E // (_NC * _NS)
_CH = 80
_NCHUNK = _EPT // _CH
_RPT = _N // _NS

_BLK = 1000
_NBLK = _N // _BLK


def _i16(v):
    return jnp.full((16,), v, dtype=jnp.int32)


def _f16(v):
    return jnp.full((16,), v, dtype=jnp.float32)


def _a_body(x_ref, w_ref, z_ref):
    z_ref[...] = jnp.dot(x_ref[...], w_ref[0], preferred_element_type=jnp.float32)


def _run_a(x, waug):
    return pl.pallas_call(
        _a_body,
        grid=(_NBLK, _NR + 1),
        in_specs=[
            pl.BlockSpec((_BLK, _D), lambda j, r: (j, 0)),
            pl.BlockSpec((1, _D, _D), lambda j, r: (r, 0, 0)),
        ],
        out_specs=pl.BlockSpec((_BLK, _D), lambda j, r: (r * _NBLK + j, 0)),
        out_shape=jax.ShapeDtypeStruct(((_NR + 1) * _N, _D), jnp.float32),
    )(x, waug)


def _b1_body(srca, dsta, ea, cnt_out, dst_v, rel_v, oh_v, zb_v, acc_s):
    c = lax.axis_index("c")
    s = lax.axis_index("s")

    @pl.loop(0, 125)
    def _zero(r):
        zb_v[r, pl.ds(0, 16)] = _f16(0.0)
        zb_v[r, pl.ds(16, 16)] = _f16(0.0)

    for j in range(5):
        pltpu.sync_copy(zb_v, acc_s.at[pl.ds(s * _RPT + j * 125, 125)])

    @pl.loop(0, _CH)
    def _zoh(r):
        oh_v[r, pl.ds(0, 16)] = _f16(0.0)
        oh_v[r, pl.ds(16, 16)] = _f16(0.0)

    plsc.subcore_barrier()

    tile_base = c * (_NS * _EPT) + s * _EPT

    @pl.loop(0, _NCHUNK)
    def _chunk(i):
        base = tile_base + i * _CH
        pltpu.sync_copy(dsta.at[pl.ds(base, _CH)], dst_v)
        pltpu.sync_copy(ea.at[pl.ds(base, _CH)], rel_v)
        iota = lax.iota(jnp.int32, 16)
        for g in range(_CH // 16):
            e16 = iota + _i16(g * 16)
            rel16 = rel_v[pl.ds(g * 16, 16)]
            plsc.store_scatter(oh_v, [e16, rel16], _f16(1.0))
        pltpu.sync_copy(oh_v, acc_s.at[dst_v], add=True)
        for g in range(_CH // 16):
            e16 = iota + _i16(g * 16)
            rel16 = rel_v[pl.ds(g * 16, 16)]
            plsc.store_scatter(oh_v, [e16, rel16], _f16(0.0))

    plsc.subcore_barrier()
    pltpu.sync_copy(acc_s.at[pl.ds(s * _RPT, _RPT)],
                    cnt_out.at[c, pl.ds(s * _RPT, _RPT)])


def _run_b1(srca, dsta, ea):
    mesh = plsc.VectorSubcoreMesh(core_axis_name="c", subcore_axis_name="s")
    return pl.kernel(
        _b1_body,
        out_type=jax.ShapeDtypeStruct((_NC, _N, _RP), jnp.float32),
        mesh=mesh,
        compiler_params=pltpu.CompilerParams(use_tc_tiling_on_sc=False, needs_layout_passes=False),
        scratch_types=[
            pltpu.VMEM((_CH,), jnp.int32),
            pltpu.VMEM((_CH,), jnp.int32),
            pltpu.VMEM((_CH, _RP), jnp.float32),
            pltpu.VMEM((125, _RP), jnp.float32),
            pltpu.VMEM_SHARED((_N, _RP), jnp.float32),
        ],
    )(srca, dsta, ea)


def _c_body(cnt_ref, r_ref):
    c = cnt_ref[0] + cnt_ref[1]
    r_ref[...] = 1.0 / jnp.maximum(c, 1.0)


def _run_c(cnt2):
    return pl.pallas_call(
        _c_body,
        grid=(_NBLK,),
        in_specs=[pl.BlockSpec((_NC, _BLK, _RP), lambda j: (0, j, 0))],
        out_specs=pl.BlockSpec((_BLK, _RP), lambda j: (j, 0)),
        out_shape=jax.ShapeDtypeStruct((_N, _RP), jnp.float32),
    )(cnt2)


def _b2_body(z2x, r2d, srca, dsta, ea, agg_out,
             gidx_all, rel_all, dst2, wrows0, wrows1, msg0, msg1, zb_v,
             acc_s, gs0, gs1, ws0, ws1, ss0, ss1):
    c = lax.axis_index("c")
    s = lax.axis_index("s")
    _DH = _D // 2

    @pl.loop(0, 125)
    def _zero(r):
        for f in range(_DH // 16):
            zb_v[r, pl.ds(f * 16, 16)] = _f16(0.0)

    for j in range(5):
        pltpu.sync_copy(zb_v, acc_s.at[pl.ds(s * _RPT + j * 125, 125)])

    plsc.subcore_barrier()

    ept = _E // _NS
    nch = ept // _CH
    tile_base = s * ept
    iota = lax.iota(jnp.int32, 16)

    pltpu.sync_copy(dsta.at[pl.ds(tile_base, ept)], gidx_all)

    @pl.loop(0, nch)
    def _bdst(i):
        for g in range(_CH // 16):
            dst2[i, pl.ds(g * 16, 16)] = gidx_all[pl.ds(i * _CH + g * 16, 16)]

    pltpu.sync_copy(srca.at[pl.ds(tile_base, ept)], gidx_all)
    pltpu.sync_copy(ea.at[pl.ds(tile_base, ept)], rel_all)

    @pl.loop(0, ept // 16)
    def _bgidx(i):
        sl = pl.ds(i * 16, 16)
        gidx_all[sl] = (rel_all[sl] * _N + gidx_all[sl]) * 2 + c

    @pl.loop(0, _CH)
    def _zmsg(r):
        for f in range(_DH // 16):
            msg0[r, pl.ds(f * 16, 16)] = _f16(0.0)
            msg1[r, pl.ds(f * 16, 16)] = _f16(0.0)

    bufs = ((msg0, wrows0, gs0, ws0, ss0), (msg1, wrows1, gs1, ws1, ss1))

    def issue(gc, b):
        msg, wrows, gs, ws, _ = bufs[b]
        gsl = gidx_all.at[pl.ds(gc * _CH, _CH)]
        pltpu.async_copy(z2x.at[gsl], msg, gs)
        pltpu.async_copy(r2d.at[dst2.at[gc]], wrows, ws)

    def wait_gathers(gc, b):
        msg, wrows, gs, ws, _ = bufs[b]
        gsl = gidx_all.at[pl.ds(gc * _CH, _CH)]
        pltpu.make_async_copy(z2x.at[gsl], msg, gs).wait()
        pltpu.make_async_copy(r2d.at[dst2.at[gc]], wrows, ws).wait()

    def scale(gc, b):
        msg, wrows, _, _, _ = bufs[b]
        for g in range(_CH // 16):
            e16 = iota + _i16(g * 16)
            rel16 = rel_all[pl.ds(gc * _CH + g * 16, 16)]
            w16 = plsc.load_gather(wrows, [e16, rel16])
            for k in range(16):
                wb = lax.gather(
                    w16, _i16(k)[:, None],
                    lax.GatherDimensionNumbers(offset_dims=(),
                                               collapsed_slice_dims=(0,),
                                               start_index_map=(0,)),
                    (1,), mode=lax.GatherScatterMode.PROMISE_IN_BOUNDS)
                e = g * 16 + k
                for f in range(_DH // 16):
                    fsl = pl.ds(f * 16, 16)
                    msg[e, fsl] = msg[e, fsl] * wb

    def issue_scatter(gc, b):
        msg, _, _, _, ss = bufs[b]
        pltpu.async_copy(msg, acc_s.at[dst2.at[gc]], ss, add=True)

    def wait_scatter(gc, b):
        msg, _, _, _, ss = bufs[b]
        pltpu.make_async_copy(msg, acc_s.at[dst2.at[gc]], ss).wait()

    issue_scatter(0, 1)
    issue(0, 0)

    @pl.loop(0, nch, step=2)
    def _chunk(g):
        wait_scatter(g, 1)
        issue(g + 1, 1)
        wait_gathers(g, 0)
        scale(g, 0)
        issue_scatter(g, 0)
        wait_scatter(g, 0)

        @pl.when(g + 2 < nch)
        def _():
            issue(g + 2, 0)

        wait_gathers(g + 1, 1)
        scale(g + 1, 1)
        issue_scatter(g + 1, 1)

    wait_scatter(nch - 1, 1)

    plsc.subcore_barrier()
    pltpu.sync_copy(acc_s.at[pl.ds(s * _RPT, _RPT)],
                    agg_out.at[c, pl.ds(s * _RPT, _RPT)])


def _run_b2(z2x, r2d, srca, dsta, ea):
    mesh = plsc.VectorSubcoreMesh(core_axis_name="c", subcore_axis_name="s")
    ept = _E // _NS
    nch = ept // _CH
    return pl.kernel(
        _b2_body,
        out_type=jax.ShapeDtypeStruct((_NC, _N, _D // 2), jnp.float32),
        mesh=mesh,
        compiler_params=pltpu.CompilerParams(use_tc_tiling_on_sc=False, needs_layout_passes=False),
        scratch_types=[
            pltpu.VMEM((ept,), jnp.int32),
            pltpu.VMEM((ept,), jnp.int32),
            pltpu.VMEM((nch, _CH), jnp.int32),
            pltpu.VMEM((_CH, _RP), jnp.float32),
            pltpu.VMEM((_CH, _RP), jnp.float32),
            pltpu.VMEM((_CH, _D // 2), jnp.float32),
            pltpu.VMEM((_CH, _D // 2), jnp.float32),
            pltpu.VMEM((125, _D // 2), jnp.float32),
            pltpu.VMEM_SHARED((_N, _D // 2), jnp.float32),
            pltpu.SemaphoreType.DMA,
            pltpu.SemaphoreType.DMA,
            pltpu.SemaphoreType.DMA,
            pltpu.SemaphoreType.DMA,
            pltpu.SemaphoreType.DMA,
            pltpu.SemaphoreType.DMA,
        ],
    )(z2x, r2d, srca, dsta, ea)


def _d1_body(agg_ref, z_ref, b1_ref, hpre_ref, st_ref):
    h = jnp.concatenate([agg_ref[0], agg_ref[1]], axis=1) + z_ref[...] + b1_ref[...]
    hpre_ref[...] = h
    s = jnp.sum(h, axis=0, keepdims=True)
    ss = jnp.sum(h * h, axis=0, keepdims=True)

    @pl.when(pl.program_id(0) == 0)
    def _():
        st_ref[...] = jnp.zeros_like(st_ref)

    st_ref[0:1, :] = st_ref[0:1, :] + s
    st_ref[1:2, :] = st_ref[1:2, :] + ss


def _run_d1(agg1, z, b1r):
    return pl.pallas_call(
        _d1_body,
        grid=(_NBLK,),
        in_specs=[
            pl.BlockSpec((_NC, _BLK, _D // 2), lambda j: (0, j, 0)),
            pl.BlockSpec((_BLK, _D), lambda j: (_NR * _NBLK + j, 0)),
            pl.BlockSpec((1, _D), lambda j: (0, 0)),
        ],
        out_specs=[
            pl.BlockSpec((_BLK, _D), lambda j: (j, 0)),
            pl.BlockSpec((8, _D), lambda j: (0, 0)),
        ],
        out_shape=[
            jax.ShapeDtypeStruct((_N, _D), jnp.float32),
            jax.ShapeDtypeStruct((8, _D), jnp.float32),
        ],
    )(agg1, z, b1r)


def _d2_body(hpre_ref, st_ref, g_ref, b_ref, w_ref, z2_ref):
    inv_n = 1.0 / _N
    m = st_ref[0:1, :] * inv_n
    var = st_ref[1:2, :] * inv_n - m * m
    inv = lax.rsqrt(var + 1e-5)
    h = (hpre_ref[...] - m) * (inv * g_ref[...]) + b_ref[...]
    h = jnp.maximum(h, 0.0)
    z2_ref[...] = jnp.dot(h, w_ref[...], preferred_element_type=jnp.float32)


def _run_d2(hpre, stats, gr, br, wcat):
    return pl.pallas_call(
        _d2_body,
        grid=(_NBLK,),
        in_specs=[
            pl.BlockSpec((_BLK, _D), lambda j: (j, 0)),
            pl.BlockSpec((8, _D), lambda j: (0, 0)),
            pl.BlockSpec((1, _D), lambda j: (0, 0)),
            pl.BlockSpec((1, _D), lambda j: (0, 0)),
            pl.BlockSpec((_D, _RP), lambda j: (0, 0)),
        ],
        out_specs=pl.BlockSpec((_BLK, _RP), lambda j: (j, 0)),
        out_shape=jax.ShapeDtypeStruct((_N, _RP), jnp.float32),
    )(hpre, stats, gr, br, wcat)


def _b4_body(z2, srca, dsta, ea, agg_out,
             src_v, dst_v, rel_v, zrows_v, msg_v, zb_v, acc_s, sem1):
    c = lax.axis_index("c")
    s = lax.axis_index("s")

    @pl.loop(0, 125)
    def _zero(r):
        zb_v[r, pl.ds(0, 16)] = _f16(0.0)
        zb_v[r, pl.ds(16, 16)] = _f16(0.0)

    for j in range(5):
        pltpu.sync_copy(zb_v, acc_s.at[pl.ds(s * _RPT + j * 125, 125)])

    @pl.loop(0, _CH)
    def _zmsg(r):
        msg_v[r, pl.ds(0, 16)] = _f16(0.0)
        msg_v[r, pl.ds(16, 16)] = _f16(0.0)

    plsc.subcore_barrier()

    tile_base = c * (_NS * _EPT) + s * _EPT
    iota = lax.iota(jnp.int32, 16)

    @pl.loop(0, _NCHUNK)
    def _chunk(i):
        base = tile_base + i * _CH
        pltpu.sync_copy(srca.at[pl.ds(base, _CH)], src_v)
        pltpu.sync_copy(dsta.at[pl.ds(base, _CH)], dst_v)
        pltpu.sync_copy(ea.at[pl.ds(base, _CH)], rel_v)
        pltpu.async_copy(z2.at[src_v], zrows_v, sem1).wait()
        for g in range(_CH // 16):
            sl = pl.ds(g * 16, 16)
            e16 = iota + _i16(g * 16)
            rel16 = rel_v[sl]
            val16 = plsc.load_gather(zrows_v, [e16, rel16])
            plsc.store_scatter(msg_v, [e16, rel16], val16)
        pltpu.sync_copy(msg_v, acc_s.at[dst_v], add=True)
        for g in range(_CH // 16):
            sl = pl.ds(g * 16, 16)
            e16 = iota + _i16(g * 16)
            plsc.store_scatter(msg_v, [e16, rel_v[sl]], _f16(0.0))

    plsc.subcore_barrier()
    pltpu.sync_copy(acc_s.at[pl.ds(s * _RPT, _RPT)],
                    agg_out.at[c, pl.ds(s * _RPT, _RPT)])


def _run_b4(z2, srca, dsta, ea):
    mesh = plsc.VectorSubcoreMesh(core_axis_name="c", subcore_axis_name="s")
    return pl.kernel(
        _b4_body,
        out_type=jax.ShapeDtypeStruct((_NC, _N, _RP), jnp.float32),
        mesh=mesh,
        compiler_params=pltpu.CompilerParams(use_tc_tiling_on_sc=False, needs_layout_passes=False),
        scratch_types=[
            pltpu.VMEM((_CH,), jnp.int32),
            pltpu.VMEM((_CH,), jnp.int32),
            pltpu.VMEM((_CH,), jnp.int32),
            pltpu.VMEM((_CH, _RP), jnp.float32),
            pltpu.VMEM((_CH, _RP), jnp.float32),
            pltpu.VMEM((125, _RP), jnp.float32),
            pltpu.VMEM_SHARED((_N, _RP), jnp.float32),
            pltpu.SemaphoreType.DMA,
        ],
    )(z2, srca, dsta, ea)


def _d3_body(agg_ref, r_ref, z2_ref, b2_ref, out_ref):
    t = (agg_ref[0] + agg_ref[1]) * r_ref[...]
    tsum = jnp.sum(t, axis=1, keepdims=True)
    out_ref[...] = tsum + z2_ref[:, _NR:_NR + 1] + b2_ref[...]


def _run_d3(agg2, r2d, z2cat, b2r):
    return pl.pallas_call(
        _d3_body,
        grid=(_NBLK,),
        in_specs=[
            pl.BlockSpec((_NC, _BLK, _RP), lambda j: (0, j, 0)),
            pl.BlockSpec((_BLK, _RP), lambda j: (j, 0)),
            pl.BlockSpec((_BLK, _RP), lambda j: (j, 0)),
            pl.BlockSpec((1, 1), lambda j: (0, 0)),
        ],
        out_specs=pl.BlockSpec((_BLK, 1), lambda j: (j, 0)),
        out_shape=jax.ShapeDtypeStruct((_N, 1), jnp.float32),
    )(agg2, r2d, z2cat, b2r)


@jax.jit
def kernel(x, edge_index, edge_attr, W1, root1, b1, gamma, beta, W2, root2, b2):
    ei = edge_index.astype(jnp.int32)
    ea = edge_attr.astype(jnp.int32)

    waug = jnp.concatenate([W1, root1[None]], axis=0)
    wcat = jnp.concatenate(
        [jnp.transpose(W2[:, :, 0]), root2,
         jnp.zeros((_D, _RP - _NR - 1), jnp.float32)], axis=1)

    z = _run_a(x, waug)
    srca = ei[0]
    dsta = ei[1]
    cnt2 = _run_b1(srca, dsta, ea)
    r2d = _run_c(cnt2)
    z2x = z.reshape(((_NR + 1) * _N * 2, _D // 2))
    agg1 = _run_b2(z2x, r2d, srca, dsta, ea)
    hpre, stats = _run_d1(agg1, z, b1[None])
    z2cat = _run_d2(hpre, stats, gamma[None], beta[None], wcat)
    agg2 = _run_b4(z2cat, srca, dsta, ea)
    out = _run_d3(agg2, r2d, z2cat, b2[None])
    return out


# speedup vs baseline: 9.4499x; 1.4007x over previous
import functools

import jax
import jax.numpy as jnp
from jax import lax
from jax.experimental import pallas as pl
from jax.experimental.pallas import tpu as pltpu
from jax.experimental.pallas import tpu_sc as plsc

_NR = 20
_N = 10000
_E = 320000
_D = 128
_RP = 32

_NC = 2
_NS = 16
_EPT = _E // (_NC * _NS)
_CH = 80
_NCHUNK = _EPT // _CH
_RPT = _N // _NS

_BLK = 1000
_NBLK = _N // _BLK


def _i16(v):
    return jnp.full((16,), v, dtype=jnp.int32)


def _f16(v):
    return jnp.full((16,), v, dtype=jnp.float32)


def _a_body(x_ref, w_ref, z_ref):
    z_ref[...] = jnp.dot(x_ref[...], w_ref[0], preferred_element_type=jnp.float32)


def _run_a(x, waug):
    return pl.pallas_call(
        _a_body,
        grid=(_NBLK, _NR + 1),
        in_specs=[
            pl.BlockSpec((_BLK, _D), lambda j, r: (j, 0)),
            pl.BlockSpec((1, _D, _D), lambda j, r: (r, 0, 0)),
        ],
        out_specs=pl.BlockSpec((_BLK, _D), lambda j, r: (r * _NBLK + j, 0)),
        out_shape=jax.ShapeDtypeStruct(((_NR + 1) * _N, _D), jnp.float32),
    )(x, waug)


def _b1_body(srca, dsta, ea, cnt_out, rel_all, dst2, oh0, oh1, zb_v,
             acc_s, ss0, ss1):
    c = lax.axis_index("c")
    s = lax.axis_index("s")

    @pl.loop(0, 125)
    def _zero(r):
        zb_v[r, pl.ds(0, 16)] = _f16(0.0)
        zb_v[r, pl.ds(16, 16)] = _f16(0.0)

    for j in range(5):
        pltpu.sync_copy(zb_v, acc_s.at[pl.ds(s * _RPT + j * 125, 125)])

    plsc.subcore_barrier()

    tile_base = c * (_NS * _EPT) + s * _EPT
    iota = lax.iota(jnp.int32, 16)

    pltpu.sync_copy(dsta.at[pl.ds(tile_base, _EPT)], rel_all)

    @pl.loop(0, _NCHUNK)
    def _bdst(i):
        for g in range(_CH // 16):
            dst2[i, pl.ds(g * 16, 16)] = rel_all[pl.ds(i * _CH + g * 16, 16)]

    pltpu.sync_copy(ea.at[pl.ds(tile_base, _EPT)], rel_all)

    @pl.loop(0, _CH)
    def _zoh(r):
        oh0[r, pl.ds(0, 16)] = _f16(0.0)
        oh0[r, pl.ds(16, 16)] = _f16(0.0)
        oh1[r, pl.ds(0, 16)] = _f16(0.0)
        oh1[r, pl.ds(16, 16)] = _f16(0.0)

    bufs = ((oh0, ss0), (oh1, ss1))

    def build(gc, b, val):
        oh, _ = bufs[b]
        for g in range(_CH // 16):
            e16 = iota + _i16(g * 16)
            rel16 = rel_all[pl.ds(gc * _CH + g * 16, 16)]
            plsc.store_scatter(oh, [e16, rel16], _f16(val))

    def issue_scatter(gc, b):
        oh, ss = bufs[b]
        pltpu.async_copy(oh, acc_s.at[dst2.at[gc]], ss, add=True)

    def wait_scatter(gc, b):
        oh, ss = bufs[b]
        pltpu.make_async_copy(oh, acc_s.at[dst2.at[gc]], ss).wait()

    issue_scatter(0, 1)

    @pl.loop(0, _NCHUNK - 1, step=2)
    def _chunk(g):
        build(g, 0, 1.0)
        wait_scatter(g, 1)

        @pl.when(g > 0)
        def _():
            build(g - 1, 1, 0.0)

        issue_scatter(g, 0)
        build(g + 1, 1, 1.0)
        wait_scatter(g, 0)
        build(g, 0, 0.0)
        issue_scatter(g + 1, 1)

    last = _NCHUNK - 1
    wait_scatter(last - 1, 1)
    build(last - 1, 1, 0.0)
    build(last, 0, 1.0)
    issue_scatter(last, 0)
    wait_scatter(last, 0)

    plsc.subcore_barrier()
    pltpu.sync_copy(acc_s.at[pl.ds(s * _RPT, _RPT)],
                    cnt_out.at[c, pl.ds(s * _RPT, _RPT)])


def _run_b1(srca, dsta, ea):
    mesh = plsc.VectorSubcoreMesh(core_axis_name="c", subcore_axis_name="s")
    return pl.kernel(
        _b1_body,
        out_type=jax.ShapeDtypeStruct((_NC, _N, _RP), jnp.float32),
        mesh=mesh,
        compiler_params=pltpu.CompilerParams(use_tc_tiling_on_sc=False, needs_layout_passes=False),
        scratch_types=[
            pltpu.VMEM((_EPT,), jnp.int32),
            pltpu.VMEM((_NCHUNK, _CH), jnp.int32),
            pltpu.VMEM((_CH, _RP), jnp.float32),
            pltpu.VMEM((_CH, _RP), jnp.float32),
            pltpu.VMEM((125, _RP), jnp.float32),
            pltpu.VMEM_SHARED((_N, _RP), jnp.float32),
            pltpu.SemaphoreType.DMA,
            pltpu.SemaphoreType.DMA,
        ],
    )(srca, dsta, ea)


def _c_body(cnt_ref, r_ref):
    c = cnt_ref[0] + cnt_ref[1]
    r_ref[...] = 1.0 / jnp.maximum(c, 1.0)


def _run_c(cnt2):
    return pl.pallas_call(
        _c_body,
        grid=(_NBLK,),
        in_specs=[pl.BlockSpec((_NC, _BLK, _RP), lambda j: (0, j, 0))],
        out_specs=pl.BlockSpec((_BLK, _RP), lambda j: (j, 0)),
        out_shape=jax.ShapeDtypeStruct((_N, _RP), jnp.float32),
    )(cnt2)


def _b2_body(z2x, r2d, srca, dsta, ea, agg_out,
             gidx_all, rel_all, dst2, wrows0, wrows1, msg0, msg1, zb_v,
             acc_s, gs0, gs1, ws0, ws1, ss0, ss1):
    c = lax.axis_index("c")
    s = lax.axis_index("s")
    _DH = _D // 2

    @pl.loop(0, 125)
    def _zero(r):
        for f in range(_DH // 16):
            zb_v[r, pl.ds(f * 16, 16)] = _f16(0.0)

    for j in range(5):
        pltpu.sync_copy(zb_v, acc_s.at[pl.ds(s * _RPT + j * 125, 125)])

    plsc.subcore_barrier()

    ept = _E // _NS
    nch = ept // _CH
    tile_base = s * ept
    iota = lax.iota(jnp.int32, 16)

    pltpu.sync_copy(dsta.at[pl.ds(tile_base, ept)], gidx_all)

    @pl.loop(0, nch)
    def _bdst(i):
        for g in range(_CH // 16):
            dst2[i, pl.ds(g * 16, 16)] = gidx_all[pl.ds(i * _CH + g * 16, 16)]

    pltpu.sync_copy(srca.at[pl.ds(tile_base, ept)], gidx_all)
    pltpu.sync_copy(ea.at[pl.ds(tile_base, ept)], rel_all)

    @pl.loop(0, ept // 16)
    def _bgidx(i):
        sl = pl.ds(i * 16, 16)
        gidx_all[sl] = (rel_all[sl] * _N + gidx_all[sl]) * 2 + c

    @pl.loop(0, _CH)
    def _zmsg(r):
        for f in range(_DH // 16):
            msg0[r, pl.ds(f * 16, 16)] = _f16(0.0)
            msg1[r, pl.ds(f * 16, 16)] = _f16(0.0)

    bufs = ((msg0, wrows0, gs0, ws0, ss0), (msg1, wrows1, gs1, ws1, ss1))

    def issue(gc, b):
        msg, wrows, gs, ws, _ = bufs[b]
        gsl = gidx_all.at[pl.ds(gc * _CH, _CH)]
        pltpu.async_copy(z2x.at[gsl], msg, gs)
        pltpu.async_copy(r2d.at[dst2.at[gc]], wrows, ws)

    def wait_gathers(gc, b):
        msg, wrows, gs, ws, _ = bufs[b]
        gsl = gidx_all.at[pl.ds(gc * _CH, _CH)]
        pltpu.make_async_copy(z2x.at[gsl], msg, gs).wait()
        pltpu.make_async_copy(r2d.at[dst2.at[gc]], wrows, ws).wait()

    def scale(gc, b):
        msg, wrows, _, _, _ = bufs[b]
        for g in range(_CH // 16):
            e16 = iota + _i16(g * 16)
            rel16 = rel_all[pl.ds(gc * _CH + g * 16, 16)]
            w16 = plsc.load_gather(wrows, [e16, rel16])
            for k in range(16):
                wb = lax.gather(
                    w16, _i16(k)[:, None],
                    lax.GatherDimensionNumbers(offset_dims=(),
                                               collapsed_slice_dims=(0,),
                                               start_index_map=(0,)),
                    (1,), mode=lax.GatherScatterMode.PROMISE_IN_BOUNDS)
                e = g * 16 + k
                for f in range(_DH // 16):
                    fsl = pl.ds(f * 16, 16)
                    msg[e, fsl] = msg[e, fsl] * wb

    def issue_scatter(gc, b):
        msg, _, _, _, ss = bufs[b]
        pltpu.async_copy(msg, acc_s.at[dst2.at[gc]], ss, add=True)

    def wait_scatter(gc, b):
        msg, _, _, _, ss = bufs[b]
        pltpu.make_async_copy(msg, acc_s.at[dst2.at[gc]], ss).wait()

    issue_scatter(0, 1)
    issue(0, 0)

    @pl.loop(0, nch, step=2)
    def _chunk(g):
        wait_scatter(g, 1)
        issue(g + 1, 1)
        wait_gathers(g, 0)
        scale(g, 0)
        issue_scatter(g, 0)
        wait_scatter(g, 0)

        @pl.when(g + 2 < nch)
        def _():
            issue(g + 2, 0)

        wait_gathers(g + 1, 1)
        scale(g + 1, 1)
        issue_scatter(g + 1, 1)

    wait_scatter(nch - 1, 1)

    plsc.subcore_barrier()
    pltpu.sync_copy(acc_s.at[pl.ds(s * _RPT, _RPT)],
                    agg_out.at[c, pl.ds(s * _RPT, _RPT)])


def _run_b2(z2x, r2d, srca, dsta, ea):
    mesh = plsc.VectorSubcoreMesh(core_axis_name="c", subcore_axis_name="s")
    ept = _E // _NS
    nch = ept // _CH
    return pl.kernel(
        _b2_body,
        out_type=jax.ShapeDtypeStruct((_NC, _N, _D // 2), jnp.float32),
        mesh=mesh,
        compiler_params=pltpu.CompilerParams(use_tc_tiling_on_sc=False, needs_layout_passes=False),
        scratch_types=[
            pltpu.VMEM((ept,), jnp.int32),
            pltpu.VMEM((ept,), jnp.int32),
            pltpu.VMEM((nch, _CH), jnp.int32),
            pltpu.VMEM((_CH, _RP), jnp.float32),
            pltpu.VMEM((_CH, _RP), jnp.float32),
            pltpu.VMEM((_CH, _D // 2), jnp.float32),
            pltpu.VMEM((_CH, _D // 2), jnp.float32),
            pltpu.VMEM((125, _D // 2), jnp.float32),
            pltpu.VMEM_SHARED((_N, _D // 2), jnp.float32),
            pltpu.SemaphoreType.DMA,
            pltpu.SemaphoreType.DMA,
            pltpu.SemaphoreType.DMA,
            pltpu.SemaphoreType.DMA,
            pltpu.SemaphoreType.DMA,
            pltpu.SemaphoreType.DMA,
        ],
    )(z2x, r2d, srca, dsta, ea)


def _d1_body(agg_ref, z_ref, b1_ref, hpre_ref, st_ref):
    h = jnp.concatenate([agg_ref[0], agg_ref[1]], axis=1) + z_ref[...] + b1_ref[...]
    hpre_ref[...] = h
    s = jnp.sum(h, axis=0, keepdims=True)
    ss = jnp.sum(h * h, axis=0, keepdims=True)

    @pl.when(pl.program_id(0) == 0)
    def _():
        st_ref[...] = jnp.zeros_like(st_ref)

    st_ref[0:1, :] = st_ref[0:1, :] + s
    st_ref[1:2, :] = st_ref[1:2, :] + ss


def _run_d1(agg1, z, b1r):
    return pl.pallas_call(
        _d1_body,
        grid=(_NBLK,),
        in_specs=[
            pl.BlockSpec((_NC, _BLK, _D // 2), lambda j: (0, j, 0)),
            pl.BlockSpec((_BLK, _D), lambda j: (_NR * _NBLK + j, 0)),
            pl.BlockSpec((1, _D), lambda j: (0, 0)),
        ],
        out_specs=[
            pl.BlockSpec((_BLK, _D), lambda j: (j, 0)),
            pl.BlockSpec((8, _D), lambda j: (0, 0)),
        ],
        out_shape=[
            jax.ShapeDtypeStruct((_N, _D), jnp.float32),
            jax.ShapeDtypeStruct((8, _D), jnp.float32),
        ],
    )(agg1, z, b1r)


def _d2_body(hpre_ref, st_ref, g_ref, b_ref, w_ref, z2_ref):
    inv_n = 1.0 / _N
    m = st_ref[0:1, :] * inv_n
    var = st_ref[1:2, :] * inv_n - m * m
    inv = lax.rsqrt(var + 1e-5)
    h = (hpre_ref[...] - m) * (inv * g_ref[...]) + b_ref[...]
    h = jnp.maximum(h, 0.0)
    z2_ref[...] = jnp.dot(h, w_ref[...], preferred_element_type=jnp.float32)


def _run_d2(hpre, stats, gr, br, wcat):
    return pl.pallas_call(
        _d2_body,
        grid=(_NBLK,),
        in_specs=[
            pl.BlockSpec((_BLK, _D), lambda j: (j, 0)),
            pl.BlockSpec((8, _D), lambda j: (0, 0)),
            pl.BlockSpec((1, _D), lambda j: (0, 0)),
            pl.BlockSpec((1, _D), lambda j: (0, 0)),
            pl.BlockSpec((_D, _RP), lambda j: (0, 0)),
        ],
        out_specs=pl.BlockSpec((_BLK, _RP), lambda j: (j, 0)),
        out_shape=jax.ShapeDtypeStruct((_N, _RP), jnp.float32),
    )(hpre, stats, gr, br, wcat)


def _b4_body(z2, srca, dsta, ea, agg_out,
             src_all, rel_all, dst2, zr0, zr1, msg0, msg1, zb_v,
             acc_s, gz0, gz1, ss0, ss1):
    c = lax.axis_index("c")
    s = lax.axis_index("s")

    @pl.loop(0, 125)
    def _zero(r):
        zb_v[r, pl.ds(0, 16)] = _f16(0.0)
        zb_v[r, pl.ds(16, 16)] = _f16(0.0)

    for j in range(5):
        pltpu.sync_copy(zb_v, acc_s.at[pl.ds(s * _RPT + j * 125, 125)])

    plsc.subcore_barrier()

    tile_base = c * (_NS * _EPT) + s * _EPT
    iota = lax.iota(jnp.int32, 16)

    pltpu.sync_copy(dsta.at[pl.ds(tile_base, _EPT)], src_all)

    @pl.loop(0, _NCHUNK)
    def _bdst(i):
        for g in range(_CH // 16):
            dst2[i, pl.ds(g * 16, 16)] = src_all[pl.ds(i * _CH + g * 16, 16)]

    pltpu.sync_copy(srca.at[pl.ds(tile_base, _EPT)], src_all)
    pltpu.sync_copy(ea.at[pl.ds(tile_base, _EPT)], rel_all)

    @pl.loop(0, _CH)
    def _zmsg(r):
        msg0[r, pl.ds(0, 16)] = _f16(0.0)
        msg0[r, pl.ds(16, 16)] = _f16(0.0)
        msg1[r, pl.ds(0, 16)] = _f16(0.0)
        msg1[r, pl.ds(16, 16)] = _f16(0.0)

    bufs = ((zr0, msg0, gz0, ss0), (zr1, msg1, gz1, ss1))

    def issue_gather(gc, b):
        zr, _, gz, _ = bufs[b]
        ssl = src_all.at[pl.ds(gc * _CH, _CH)]
        pltpu.async_copy(z2.at[ssl], zr, gz)

    def wait_gather(gc, b):
        zr, _, gz, _ = bufs[b]
        ssl = src_all.at[pl.ds(gc * _CH, _CH)]
        pltpu.make_async_copy(z2.at[ssl], zr, gz).wait()

    def build(gc, b):
        zr, msg, _, _ = bufs[b]
        for g in range(_CH // 16):
            e16 = iota + _i16(g * 16)
            rel16 = rel_all[pl.ds(gc * _CH + g * 16, 16)]
            val16 = plsc.load_gather(zr, [e16, rel16])
            plsc.store_scatter(msg, [e16, rel16], val16)

    def rezero(gc, b):
        _, msg, _, _ = bufs[b]
        for g in range(_CH // 16):
            e16 = iota + _i16(g * 16)
            rel16 = rel_all[pl.ds(gc * _CH + g * 16, 16)]
            plsc.store_scatter(msg, [e16, rel16], _f16(0.0))

    def issue_scatter(gc, b):
        _, msg, _, ss = bufs[b]
        pltpu.async_copy(msg, acc_s.at[dst2.at[gc]], ss, add=True)

    def wait_scatter(gc, b):
        _, msg, _, ss = bufs[b]
        pltpu.make_async_copy(msg, acc_s.at[dst2.at[gc]], ss).wait()

    issue_scatter(0, 1)
    issue_gather(0, 0)

    @pl.loop(0, _NCHUNK - 1, step=2)
    def _chunk(g):
        wait_scatter(g, 1)

        @pl.when(g > 0)
        def _():
            rezero(g - 1, 1)

        issue_gather(g + 1, 1)
        wait_gather(g, 0)
        build(g, 0)
        issue_scatter(g, 0)
        wait_scatter(g, 0)
        rezero(g, 0)

        @pl.when(g + 2 < _NCHUNK)
        def _():
            issue_gather(g + 2, 0)

        wait_gather(g + 1, 1)
        build(g + 1, 1)
        issue_scatter(g + 1, 1)

    last = _NCHUNK - 1
    wait_scatter(last - 1, 1)
    wait_gather(last, 0)
    build(last, 0)
    issue_scatter(last, 0)
    wait_scatter(last, 0)

    plsc.subcore_barrier()
    pltpu.sync_copy(acc_s.at[pl.ds(s * _RPT, _RPT)],
                    agg_out.at[c, pl.ds(s * _RPT, _RPT)])


def _run_b4(z2, srca, dsta, ea):
    mesh = plsc.VectorSubcoreMesh(core_axis_name="c", subcore_axis_name="s")
    return pl.kernel(
        _b4_body,
        out_type=jax.ShapeDtypeStruct((_NC, _N, _RP), jnp.float32),
        mesh=mesh,
        compiler_params=pltpu.CompilerParams(use_tc_tiling_on_sc=False, needs_layout_passes=False),
        scratch_types=[
            pltpu.VMEM((_EPT,), jnp.int32),
            pltpu.VMEM((_EPT,), jnp.int32),
            pltpu.VMEM((_NCHUNK, _CH), jnp.int32),
            pltpu.VMEM((_CH, _RP), jnp.float32),
            pltpu.VMEM((_CH, _RP), jnp.float32),
            pltpu.VMEM((_CH, _RP), jnp.float32),
            pltpu.VMEM((_CH, _RP), jnp.float32),
            pltpu.VMEM((125, _RP), jnp.float32),
            pltpu.VMEM_SHARED((_N, _RP), jnp.float32),
            pltpu.SemaphoreType.DMA,
            pltpu.SemaphoreType.DMA,
            pltpu.SemaphoreType.DMA,
            pltpu.SemaphoreType.DMA,
        ],
    )(z2, srca, dsta, ea)


def _d3_body(agg_ref, r_ref, z2_ref, b2_ref, out_ref):
    t = (agg_ref[0] + agg_ref[1]) * r_ref[...]
    tsum = jnp.sum(t, axis=1, keepdims=True)
    out_ref[...] = tsum + z2_ref[:, _NR:_NR + 1] + b2_ref[...]


def _run_d3(agg2, r2d, z2cat, b2r):
    return pl.pallas_call(
        _d3_body,
        grid=(_NBLK,),
        in_specs=[
            pl.BlockSpec((_NC, _BLK, _RP), lambda j: (0, j, 0)),
            pl.BlockSpec((_BLK, _RP), lambda j: (j, 0)),
            pl.BlockSpec((_BLK, _RP), lambda j: (j, 0)),
            pl.BlockSpec((1, 1), lambda j: (0, 0)),
        ],
        out_specs=pl.BlockSpec((_BLK, 1), lambda j: (j, 0)),
        out_shape=jax.ShapeDtypeStruct((_N, 1), jnp.float32),
    )(agg2, r2d, z2cat, b2r)


@jax.jit
def kernel(x, edge_index, edge_attr, W1, root1, b1, gamma, beta, W2, root2, b2):
    ei = edge_index.astype(jnp.int32)
    ea = edge_attr.astype(jnp.int32)

    waug = jnp.concatenate([W1, root1[None]], axis=0)
    wcat = jnp.concatenate(
        [jnp.transpose(W2[:, :, 0]), root2,
         jnp.zeros((_D, _RP - _NR - 1), jnp.float32)], axis=1)

    z = _run_a(x, waug)
    srca = ei[0]
    dsta = ei[1]
    cnt2 = _run_b1(srca, dsta, ea)
    r2d = _run_c(cnt2)
    z2x = z.reshape(((_NR + 1) * _N * 2, _D // 2))
    agg1 = _run_b2(z2x, r2d, srca, dsta, ea)
    hpre, stats = _run_d1(agg1, z, b1[None])
    z2cat = _run_d2(hpre, stats, gamma[None], beta[None], wcat)
    agg2 = _run_b4(z2cat, srca, dsta, ea)
    out = _run_d3(agg2, r2d, z2cat, b2[None])
    return out
